# baseline (speedup 1.0000x reference)
import functools

import jax
import jax.numpy as jnp
from jax import lax
from jax.experimental import pallas as pl
from jax.experimental.pallas import tpu as pltpu
from jax.experimental.pallas import tpu_sc as plsc

N_NODES = 10000
N_EDGES = 320000
D_NODE = 128
D_EDGE = 16
HIDDEN = 128
N_GRAPHS = 64

NC = 2
NS = 16
NW = NC * NS
K = 128
IDXROWS = N_EDGES // K
IDXPAD = 2560
RPT = 624
RTAIL = N_NODES - NS * RPT
ZR = 104
NZB = RPT // ZR

ROWS_A = 1280
SEG_FULL = (0, IDXROWS, IDXPAD // NW)
SEG_A = (0, ROWS_A, ROWS_A // NW)
SEG_B = (ROWS_A, IDXROWS - ROWS_A, (IDXPAD - ROWS_A) // NW)
E_A = ROWS_A * K

BE = 2560
GRID_A = E_A // BE
GRID_B = (N_EDGES - E_A) // BE
GRID_E = N_EDGES // BE
BN = 2000
GRID_N = N_NODES // BN


def _mesh():
    return plsc.VectorSubcoreMesh(core_axis_name="c", subcore_axis_name="s")


def _worker_bounds(c, s, seg):
    row0, valid, rpw = seg
    w = s * NC + c
    irow = row0 + w * rpw
    npair = jnp.clip((valid - w * rpw) // 2, 0, rpw // 2)
    return w, irow, npair


def _sc_gather_body(seg, table, idx2d, out, idxb, bufa, bufb, gsa, gsb, ssa, ssb):
    c = lax.axis_index("c")
    s = lax.axis_index("s")
    w, irow, npair = _worker_bounds(c, s, seg)
    rpw = seg[2]
    pltpu.sync_copy(idx2d.at[pl.ds(pl.multiple_of(irow, 8), rpw)], idxb)

    def ebase(j):
        return pl.multiple_of((irow + j) * K, K)

    def start_gather(j, buf, sem):
        pltpu.async_copy(table.at[idxb.at[j]], buf, sem)

    def start_store(j, buf, sem):
        pltpu.async_copy(buf, out.at[pl.ds(ebase(j), K)], sem)

    def drain_gather(buf, sem):
        pltpu.make_async_copy(table.at[pl.ds(0, K)], buf, sem).wait()

    def drain_store(buf, sem):
        pltpu.make_async_copy(buf, out.at[pl.ds(0, K)], sem).wait()

    @pl.when(npair > 0)
    def _run():
        start_gather(0, bufa, gsa)

        def pair(j2, carry):
            p = 2 * j2
            q = p + 1

            @pl.when(j2 > 0)
            def _():
                drain_store(bufb, ssb)

            start_gather(q, bufb, gsb)
            drain_gather(bufa, gsa)
            start_store(p, bufa, ssa)

            @pl.when(j2 < npair - 1)
            def _():
                drain_store(bufa, ssa)
                start_gather(p + 2, bufa, gsa)

            drain_gather(bufb, gsb)
            start_store(q, bufb, ssb)
            return carry

        lax.fori_loop(0, npair, pair, 0)
        drain_store(bufa, ssa)
        drain_store(bufb, ssb)


def _sc_gather(table, idx2d, seg=SEG_FULL):
    rpw = seg[2]
    f = pl.kernel(
        functools.partial(_sc_gather_body, seg),
        out_type=jax.ShapeDtypeStruct((N_EDGES, HIDDEN), table.dtype),
        mesh=_mesh(),
        scratch_types=[
            pltpu.VMEM((rpw, K), jnp.int32),
            pltpu.VMEM((K, HIDDEN), table.dtype),
            pltpu.VMEM((K, HIDDEN), table.dtype),
            pltpu.SemaphoreType.DMA,
            pltpu.SemaphoreType.DMA,
            pltpu.SemaphoreType.DMA,
            pltpu.SemaphoreType.DMA,
        ],
    )
    return f(table, idx2d)


def _sc_scatter_body(seg, vals, idx2d, zrows, out, idxb, bufa, bufb, vsa, vsb, acc):
    c = lax.axis_index("c")
    s = lax.axis_index("s")
    w, irow, npair = _worker_bounds(c, s, seg)
    rpw = seg[2]
    rbase = s * RPT

    pltpu.sync_copy(idx2d.at[pl.ds(pl.multiple_of(irow, 8), rpw)], idxb)
    pltpu.sync_copy(zrows, bufa.at[pl.ds(0, ZR)])
    for j in range(NZB):
        pltpu.sync_copy(bufa.at[pl.ds(0, ZR)], acc.at[pl.ds(rbase + j * ZR, ZR)])

    @pl.when(s == NS - 1)
    def _zero_tail():
        pltpu.sync_copy(bufa.at[pl.ds(0, RTAIL)], acc.at[pl.ds(NS * RPT, RTAIL)])

    plsc.subcore_barrier()

    def ebase(j):
        return pl.multiple_of((irow + j) * K, K)

    def start_load(j, buf, sem):
        pltpu.async_copy(vals.at[pl.ds(ebase(j), K)], buf, sem)

    def drain_load(buf, sem):
        pltpu.make_async_copy(vals.at[pl.ds(0, K)], buf, sem).wait()

    @pl.when(npair > 0)
    def _run():
        start_load(0, bufa, vsa)

        def pair(j2, carry):
            p = 2 * j2
            q = p + 1
            start_load(q, bufb, vsb)
            drain_load(bufa, vsa)
            pltpu.sync_copy(bufa, acc.at[idxb.at[p]], add=True)

            @pl.when(j2 < npair - 1)
            def _():
                start_load(p + 2, bufa, vsa)

            drain_load(bufb, vsb)
            pltpu.sync_copy(bufb, acc.at[idxb.at[q]], add=True)
            return carry

        lax.fori_loop(0, npair, pair, 0)

    plsc.subcore_barrier()

    for j in range(NZB):
        pltpu.sync_copy(acc.at[pl.ds(rbase + j * ZR, ZR)], bufa.at[pl.ds(0, ZR)])
        pltpu.sync_copy(bufa.at[pl.ds(0, ZR)], out.at[c, pl.ds(rbase + j * ZR, ZR)])

    @pl.when(s == NS - 1)
    def _write_tail():
        pltpu.sync_copy(acc.at[pl.ds(NS * RPT, RTAIL)], bufa.at[pl.ds(0, RTAIL)])
        pltpu.sync_copy(bufa.at[pl.ds(0, RTAIL)], out.at[c, pl.ds(NS * RPT, RTAIL)])


def _sc_scatter(vals, idx2d, zrows, seg=SEG_FULL):
    rpw = seg[2]
    f = pl.kernel(
        functools.partial(_sc_scatter_body, seg),
        out_type=jax.ShapeDtypeStruct((NC, N_NODES, HIDDEN), jnp.float32),
        mesh=_mesh(),
        scratch_types=[
            pltpu.VMEM((rpw, K), jnp.int32),
            pltpu.VMEM((K, HIDDEN), jnp.float32),
            pltpu.VMEM((K, HIDDEN), jnp.float32),
            pltpu.SemaphoreType.DMA,
            pltpu.SemaphoreType.DMA,
            pltpu.VMEM_SHARED((N_NODES, HIDDEN), jnp.float32),
        ],
    )
    return f(vals, idx2d, zrows)


def _pair_swap(y):
    even = (lax.broadcasted_iota(jnp.int32, y.shape, 0) & 1) == 0
    return jnp.where(even, pltpu.roll(y, y.shape[0] - 1, 0),
                     pltpu.roll(y, 1, 0))


def _tc_init_body(xg_ref, ea_ref, Wx_ref, We_ref, b_ref, W1_ref, h0_ref, y_ref):
    h0 = jnp.maximum(
        jnp.dot(xg_ref[...].astype(jnp.float32), Wx_ref[...],
                preferred_element_type=jnp.float32)
        + jnp.dot(ea_ref[...].astype(jnp.float32), We_ref[...],
                  preferred_element_type=jnp.float32)
        + b_ref[...],
        0.0,
    )
    h0_ref[...] = h0.astype(jnp.bfloat16)
    y_ref[...] = jnp.dot(h0, W1_ref[...], preferred_element_type=jnp.float32)


def _tc_init(xg, ea, Wxt, Wet, bi, W1t, grid, off):
    return pl.pallas_call(
        _tc_init_body,
        grid=(grid,),
        in_specs=[
            pl.BlockSpec((BE, D_NODE), _eoff(off)),
            pl.BlockSpec((BE, D_EDGE), _eoff(off)),
            pl.BlockSpec((D_NODE, HIDDEN), _wmap),
            pl.BlockSpec((D_EDGE, HIDDEN), _wmap),
            pl.BlockSpec((1, HIDDEN), _wmap),
            pl.BlockSpec((HIDDEN, HIDDEN), _wmap),
        ],
        out_specs=[
            pl.BlockSpec((BE, HIDDEN), _eoff(off)),
            pl.BlockSpec((BE, HIDDEN), _eoff(off)),
        ],
        out_shape=[
            jax.ShapeDtypeStruct((N_EDGES, HIDDEN), jnp.bfloat16),
            jax.ShapeDtypeStruct((N_EDGES, HIDDEN), jnp.float32),
        ],
    )(xg, ea, Wxt, Wet, bi, W1t)


def _tc_conv_body(g_ref, y_ref, h0_ref, b_ref, W_ref, out_ref):
    h = jnp.maximum(
        g_ref[...] - _pair_swap(y_ref[...]) + b_ref[...]
        + h0_ref[...].astype(jnp.float32), 0.0)
    out_ref[...] = jnp.dot(h, W_ref[...], preferred_element_type=jnp.float32)


def _eoff(off):
    return lambda i: (i + off, 0)


def _wmap(i):
    return (0, 0)


def _tc_conv(g, y, h0, b, Wt, grid, off):
    return pl.pallas_call(
        _tc_conv_body,
        grid=(grid,),
        in_specs=[
            pl.BlockSpec((BE, HIDDEN), _eoff(off)),
            pl.BlockSpec((BE, HIDDEN), _eoff(off)),
            pl.BlockSpec((BE, HIDDEN), _eoff(off)),
            pl.BlockSpec((1, HIDDEN), _wmap),
            pl.BlockSpec((HIDDEN, HIDDEN), _wmap),
        ],
        out_specs=pl.BlockSpec((BE, HIDDEN), _eoff(off)),
        out_shape=jax.ShapeDtypeStruct((N_EDGES, HIDDEN), jnp.float32),
    )(g, y, h0, b, Wt)


def _tc_convlast_body(g_ref, y_ref, h0_ref, b_ref, out_ref):
    out_ref[...] = jnp.maximum(
        g_ref[...] - _pair_swap(y_ref[...]) + b_ref[...]
        + h0_ref[...].astype(jnp.float32), 0.0)


def _tc_convlast(g, y, h0, b, grid, off):
    return pl.pallas_call(
        _tc_convlast_body,
        grid=(grid,),
        in_specs=[
            pl.BlockSpec((BE, HIDDEN), _eoff(off)),
            pl.BlockSpec((BE, HIDDEN), _eoff(off)),
            pl.BlockSpec((BE, HIDDEN), _eoff(off)),
            pl.BlockSpec((1, HIDDEN), _wmap),
        ],
        out_specs=pl.BlockSpec((BE, HIDDEN), _eoff(off)),
        out_shape=jax.ShapeDtypeStruct((N_EDGES, HIDDEN), jnp.float32),
    )(g, y, h0, b)


def _tc_add4_body(a_ref, b_ref, c_ref, d_ref, o_ref):
    o_ref[...] = (a_ref[...] + b_ref[...]) + (c_ref[...] + d_ref[...])


def _tc_add4(a, b, c, d):
    spec = pl.BlockSpec((BN, HIDDEN), lambda i: (i, 0))
    return pl.pallas_call(
        _tc_add4_body,
        grid=(GRID_N,),
        in_specs=[spec, spec, spec, spec],
        out_specs=spec,
        out_shape=jax.ShapeDtypeStruct((N_NODES, HIDDEN), jnp.float32),
    )(a, b, c, d)


def _tc_final_body(x_ref, pa0_ref, pa1_ref, pb0_ref, pb1_ref, bt_ref,
                   At_ref, Bt_ref, be_ref, out_ref):
    s = (pa0_ref[...] + pa1_ref[...]) + (pb0_ref[...] + pb1_ref[...])
    hn = jnp.maximum(
        jnp.dot(x_ref[...], At_ref[...], preferred_element_type=jnp.float32)
        + jnp.dot(s, Bt_ref[...], preferred_element_type=jnp.float32)
        + be_ref[...],
        0.0,
    )
    oh = (bt_ref[...] == lax.broadcasted_iota(jnp.int32, (BN, N_GRAPHS), 1)
          ).astype(jnp.float32)
    part = lax.dot_general(oh, hn, (((0,), (0,)), ((), ())),
                           preferred_element_type=jnp.float32)

    @pl.when(pl.program_id(0) == 0)
    def _():
        out_ref[...] = jnp.zeros_like(out_ref)

    out_ref[...] += part


def _tc_final(x, pa0, pa1, pb0, pb1, bt, At, Bt, be):
    nspec = pl.BlockSpec((BN, HIDDEN), lambda i: (i, 0))
    return pl.pallas_call(
        _tc_final_body,
        grid=(GRID_N,),
        in_specs=[
            pl.BlockSpec((BN, D_NODE), lambda i: (i, 0)),
            nspec, nspec, nspec, nspec,
            pl.BlockSpec((BN, 1), lambda i: (i, 0)),
            pl.BlockSpec((D_NODE, HIDDEN), _wmap),
            pl.BlockSpec((HIDDEN, HIDDEN), _wmap),
            pl.BlockSpec((1, HIDDEN), _wmap),
        ],
        out_specs=pl.BlockSpec((N_GRAPHS, HIDDEN), lambda i: (0, 0)),
        out_shape=jax.ShapeDtypeStruct((N_GRAPHS, HIDDEN), jnp.float32),
    )(x, pa0, pa1, pb0, pb1, bt, At, Bt, be)


def kernel(x, edge_index, edge_attr, batch, W_init, b_init, W1, b1, W2, b2,
           W3, b3, W_e2n, b_e2n):
    row = edge_index[0].astype(jnp.int32)
    col = edge_index[1].astype(jnp.int32)
    pad = ((0, IDXPAD - IDXROWS), (0, 0))
    row2d = jnp.pad(row.reshape(IDXROWS, K), pad)
    col2d = jnp.pad(col.reshape(IDXROWS, K), pad)

    Wxt = W_init[:, :D_NODE].T
    Wet = W_init[:, D_NODE:].T
    Wts = (W1.T, W2.T, W3.T)
    bis = (b1[None, :], b2[None, :], b3[None, :])
    zrows = jnp.zeros((ZR, HIDDEN), jnp.float32)

    xg = _sc_gather(x, row2d)
    h0, y = _tc_init(xg, edge_attr.astype(jnp.bfloat16), Wxt, Wet,
                     b_init[None, :], Wts[0], GRID_E, 0)

    y_lo = y_hi = y
    h3_lo = h3_hi = None
    for i in range(3):
        part_a = _sc_scatter(y_lo, col2d, zrows, SEG_A)
        part_b = _sc_scatter(y_hi, col2d, zrows, SEG_B)
        agg = _tc_add4(part_a[0], part_a[1], part_b[0], part_b[1])
        g_a = _sc_gather(agg, row2d, SEG_A)
        g_b = _sc_gather(agg, row2d, SEG_B)
        if i < 2:
            y_lo = _tc_conv(g_a, y_lo, h0, bis[i], Wts[i + 1], GRID_A, 0)
            y_hi = _tc_conv(g_b, y_hi, h0, bis[i], Wts[i + 1], GRID_B, GRID_A)
        else:
            h3_lo = _tc_convlast(g_a, y_lo, h0, bis[i], GRID_A, 0)
            h3_hi = _tc_convlast(g_b, y_hi, h0, bis[i], GRID_B, GRID_A)

    part_a = _sc_scatter(h3_lo, col2d, zrows, SEG_A)
    part_b = _sc_scatter(h3_hi, col2d, zrows, SEG_B)
    bt = batch.astype(jnp.int32).reshape(N_NODES, 1)
    emb = _tc_final(x, part_a[0], part_a[1], part_b[0], part_b[1], bt,
                    W_e2n[:, :D_NODE].T, W_e2n[:, D_NODE:].T, b_e2n[None, :])
    return emb


# speedup vs baseline: 1.0325x; 1.0325x over previous
import functools

import jax
import jax.numpy as jnp
from jax import lax
from jax.experimental import pallas as pl
from jax.experimental.pallas import tpu as pltpu
from jax.experimental.pallas import tpu_sc as plsc

N_NODES = 10000
N_EDGES = 320000
D_NODE = 128
D_EDGE = 16
HIDDEN = 128
N_GRAPHS = 64

NC = 2
NS = 16
NW = NC * NS
K = 128
IDXROWS = N_EDGES // K
IDXPAD = 2560
RPT = 624
RTAIL = N_NODES - NS * RPT
ZR = 104
NZB = RPT // ZR

ROWS_A = 1280
SEG_FULL = (0, IDXROWS, IDXPAD // NW)
SEG_A = (0, ROWS_A, ROWS_A // NW)
SEG_B = (ROWS_A, IDXROWS - ROWS_A, (IDXPAD - ROWS_A) // NW)
E_A = ROWS_A * K

BE = 2560
GRID_A = E_A // BE
GRID_B = (N_EDGES - E_A) // BE
GRID_E = N_EDGES // BE
BN = 2000
GRID_N = N_NODES // BN


def _mesh():
    return plsc.VectorSubcoreMesh(core_axis_name="c", subcore_axis_name="s")


def _worker_bounds(c, s, seg):
    row0, valid, rpw = seg
    w = s * NC + c
    irow = row0 + w * rpw
    npair = jnp.clip((valid - w * rpw) // 2, 0, rpw // 2)
    return w, irow, npair


def _sc_gather_body(seg, table, idx2d, out, idxb, bufa, bufb, gsa, gsb, ssa, ssb):
    c = lax.axis_index("c")
    s = lax.axis_index("s")
    w, irow, npair = _worker_bounds(c, s, seg)
    rpw = seg[2]
    pltpu.sync_copy(idx2d.at[pl.ds(pl.multiple_of(irow, 8), rpw)], idxb)

    def ebase(j):
        return pl.multiple_of((irow + j) * K, K)

    def start_gather(j, buf, sem):
        pltpu.async_copy(table.at[idxb.at[j]], buf, sem)

    def start_store(j, buf, sem):
        pltpu.async_copy(buf, out.at[pl.ds(ebase(j), K)], sem)

    def drain_gather(buf, sem):
        pltpu.make_async_copy(table.at[pl.ds(0, K)], buf, sem).wait()

    def drain_store(buf, sem):
        pltpu.make_async_copy(buf, out.at[pl.ds(0, K)], sem).wait()

    @pl.when(npair > 0)
    def _run():
        start_gather(0, bufa, gsa)

        def pair(j2, carry):
            p = 2 * j2
            q = p + 1

            @pl.when(j2 > 0)
            def _():
                drain_store(bufb, ssb)

            start_gather(q, bufb, gsb)
            drain_gather(bufa, gsa)
            start_store(p, bufa, ssa)

            @pl.when(j2 < npair - 1)
            def _():
                drain_store(bufa, ssa)
                start_gather(p + 2, bufa, gsa)

            drain_gather(bufb, gsb)
            start_store(q, bufb, ssb)
            return carry

        lax.fori_loop(0, npair, pair, 0)
        drain_store(bufa, ssa)
        drain_store(bufb, ssb)


def _sc_gather(table, idx2d, seg=SEG_FULL):
    rpw = seg[2]
    f = pl.kernel(
        functools.partial(_sc_gather_body, seg),
        out_type=jax.ShapeDtypeStruct((N_EDGES, HIDDEN), table.dtype),
        mesh=_mesh(),
        scratch_types=[
            pltpu.VMEM((rpw, K), jnp.int32),
            pltpu.VMEM((K, HIDDEN), table.dtype),
            pltpu.VMEM((K, HIDDEN), table.dtype),
            pltpu.SemaphoreType.DMA,
            pltpu.SemaphoreType.DMA,
            pltpu.SemaphoreType.DMA,
            pltpu.SemaphoreType.DMA,
        ],
    )
    return f(table, idx2d)


def _sc_scatter_body(seg, vals, idx2d, zrows, out, idxb, bufa, bufb, vsa, vsb, acc):
    c = lax.axis_index("c")
    s = lax.axis_index("s")
    w, irow, npair = _worker_bounds(c, s, seg)
    rpw = seg[2]
    rbase = s * RPT

    pltpu.sync_copy(idx2d.at[pl.ds(pl.multiple_of(irow, 8), rpw)], idxb)
    pltpu.sync_copy(zrows, bufa.at[pl.ds(0, ZR)])
    for j in range(NZB):
        pltpu.sync_copy(bufa.at[pl.ds(0, ZR)], acc.at[pl.ds(rbase + j * ZR, ZR)])

    @pl.when(s == NS - 1)
    def _zero_tail():
        pltpu.sync_copy(bufa.at[pl.ds(0, RTAIL)], acc.at[pl.ds(NS * RPT, RTAIL)])

    plsc.subcore_barrier()

    def ebase(j):
        return pl.multiple_of((irow + j) * K, K)

    def start_load(j, buf, sem):
        pltpu.async_copy(vals.at[pl.ds(ebase(j), K)], buf, sem)

    def drain_load(buf, sem):
        pltpu.make_async_copy(vals.at[pl.ds(0, K)], buf, sem).wait()

    @pl.when(npair > 0)
    def _run():
        start_load(0, bufa, vsa)

        def pair(j2, carry):
            p = 2 * j2
            q = p + 1
            start_load(q, bufb, vsb)
            drain_load(bufa, vsa)
            pltpu.sync_copy(bufa, acc.at[idxb.at[p]], add=True)

            @pl.when(j2 < npair - 1)
            def _():
                start_load(p + 2, bufa, vsa)

            drain_load(bufb, vsb)
            pltpu.sync_copy(bufb, acc.at[idxb.at[q]], add=True)
            return carry

        lax.fori_loop(0, npair, pair, 0)

    plsc.subcore_barrier()

    for j in range(NZB):
        pltpu.sync_copy(acc.at[pl.ds(rbase + j * ZR, ZR)], bufa.at[pl.ds(0, ZR)])
        pltpu.sync_copy(bufa.at[pl.ds(0, ZR)], out.at[c, pl.ds(rbase + j * ZR, ZR)])

    @pl.when(s == NS - 1)
    def _write_tail():
        pltpu.sync_copy(acc.at[pl.ds(NS * RPT, RTAIL)], bufa.at[pl.ds(0, RTAIL)])
        pltpu.sync_copy(bufa.at[pl.ds(0, RTAIL)], out.at[c, pl.ds(NS * RPT, RTAIL)])


def _sc_scatter(vals, idx2d, zrows, seg=SEG_FULL):
    rpw = seg[2]
    f = pl.kernel(
        functools.partial(_sc_scatter_body, seg),
        out_type=jax.ShapeDtypeStruct((NC, N_NODES, HIDDEN), jnp.float32),
        mesh=_mesh(),
        scratch_types=[
            pltpu.VMEM((rpw, K), jnp.int32),
            pltpu.VMEM((K, HIDDEN), jnp.float32),
            pltpu.VMEM((K, HIDDEN), jnp.float32),
            pltpu.SemaphoreType.DMA,
            pltpu.SemaphoreType.DMA,
            pltpu.VMEM_SHARED((N_NODES, HIDDEN), jnp.float32),
        ],
    )
    return f(vals, idx2d, zrows)


def _pair_swap(y):
    even = (lax.broadcasted_iota(jnp.int32, y.shape, 0) & 1) == 0
    return jnp.where(even, pltpu.roll(y, y.shape[0] - 1, 0),
                     pltpu.roll(y, 1, 0))


def _tc_init_body(xg_ref, ea_ref, Wx_ref, We_ref, b_ref, W1_ref, h0_ref, y_ref):
    h0 = jnp.maximum(
        jnp.dot(xg_ref[...].astype(jnp.float32), Wx_ref[...],
                preferred_element_type=jnp.float32)
        + jnp.dot(ea_ref[...].astype(jnp.float32), We_ref[...],
                  preferred_element_type=jnp.float32)
        + b_ref[...],
        0.0,
    )
    h0_ref[...] = h0.astype(jnp.bfloat16)
    y_ref[...] = jnp.dot(h0, W1_ref[...], preferred_element_type=jnp.float32)


def _tc_init(xg, ea, Wxt, Wet, bi, W1t, grid, off):
    return pl.pallas_call(
        _tc_init_body,
        grid=(grid,),
        in_specs=[
            pl.BlockSpec((BE, D_NODE), _eoff(off)),
            pl.BlockSpec((BE, D_EDGE), _eoff(off)),
            pl.BlockSpec((D_NODE, HIDDEN), _wmap),
            pl.BlockSpec((D_EDGE, HIDDEN), _wmap),
            pl.BlockSpec((1, HIDDEN), _wmap),
            pl.BlockSpec((HIDDEN, HIDDEN), _wmap),
        ],
        out_specs=[
            pl.BlockSpec((BE, HIDDEN), _eoff(off)),
            pl.BlockSpec((BE, HIDDEN), _eoff(off)),
        ],
        out_shape=[
            jax.ShapeDtypeStruct((N_EDGES, HIDDEN), jnp.bfloat16),
            jax.ShapeDtypeStruct((N_EDGES, HIDDEN), jnp.float32),
        ],
    )(xg, ea, Wxt, Wet, bi, W1t)


def _tc_conv_body(g_ref, y_ref, h0_ref, b_ref, W_ref, out_ref):
    h = jnp.maximum(
        g_ref[...] - _pair_swap(y_ref[...]) + b_ref[...]
        + h0_ref[...].astype(jnp.float32), 0.0)
    out_ref[...] = jnp.dot(h, W_ref[...], preferred_element_type=jnp.float32)


def _eoff(off):
    return lambda i: (i + off, 0)


def _wmap(i):
    return (0, 0)


def _tc_conv(g, y, h0, b, Wt, grid, off):
    return pl.pallas_call(
        _tc_conv_body,
        grid=(grid,),
        in_specs=[
            pl.BlockSpec((BE, HIDDEN), _eoff(off)),
            pl.BlockSpec((BE, HIDDEN), _eoff(off)),
            pl.BlockSpec((BE, HIDDEN), _eoff(off)),
            pl.BlockSpec((1, HIDDEN), _wmap),
            pl.BlockSpec((HIDDEN, HIDDEN), _wmap),
        ],
        out_specs=pl.BlockSpec((BE, HIDDEN), _eoff(off)),
        out_shape=jax.ShapeDtypeStruct((N_EDGES, HIDDEN), jnp.float32),
    )(g, y, h0, b, Wt)


def _tc_convlast_body(g_ref, y_ref, h0_ref, b_ref, out_ref):
    out_ref[...] = jnp.maximum(
        g_ref[...] - _pair_swap(y_ref[...]) + b_ref[...]
        + h0_ref[...].astype(jnp.float32), 0.0)


def _tc_convlast(g, y, h0, b, grid, off):
    return pl.pallas_call(
        _tc_convlast_body,
        grid=(grid,),
        in_specs=[
            pl.BlockSpec((BE, HIDDEN), _eoff(off)),
            pl.BlockSpec((BE, HIDDEN), _eoff(off)),
            pl.BlockSpec((BE, HIDDEN), _eoff(off)),
            pl.BlockSpec((1, HIDDEN), _wmap),
        ],
        out_specs=pl.BlockSpec((BE, HIDDEN), _eoff(off)),
        out_shape=jax.ShapeDtypeStruct((N_EDGES, HIDDEN), jnp.float32),
    )(g, y, h0, b)


def _tc_add4_body(a_ref, b_ref, c_ref, d_ref, o_ref):
    o_ref[...] = (a_ref[...] + b_ref[...]) + (c_ref[...] + d_ref[...])


def _tc_add4(a, b, c, d):
    spec = pl.BlockSpec((BN, HIDDEN), lambda i: (i, 0))
    return pl.pallas_call(
        _tc_add4_body,
        grid=(GRID_N,),
        in_specs=[spec, spec, spec, spec],
        out_specs=spec,
        out_shape=jax.ShapeDtypeStruct((N_NODES, HIDDEN), jnp.float32),
    )(a, b, c, d)


def _tc_final_body(x_ref, pa0_ref, pa1_ref, pb0_ref, pb1_ref, bt_ref,
                   At_ref, Bt_ref, be_ref, out_ref):
    s = (pa0_ref[...] + pa1_ref[...]) + (pb0_ref[...] + pb1_ref[...])
    hn = jnp.maximum(
        jnp.dot(x_ref[...], At_ref[...], preferred_element_type=jnp.float32)
        + jnp.dot(s, Bt_ref[...], preferred_element_type=jnp.float32)
        + be_ref[...],
        0.0,
    )
    oh = (bt_ref[...] == lax.broadcasted_iota(jnp.int32, (BN, N_GRAPHS), 1)
          ).astype(jnp.float32)
    part = lax.dot_general(oh, hn, (((0,), (0,)), ((), ())),
                           preferred_element_type=jnp.float32)

    @pl.when(pl.program_id(0) == 0)
    def _():
        out_ref[...] = jnp.zeros_like(out_ref)

    out_ref[...] += part


def _tc_final(x, pa0, pa1, pb0, pb1, bt, At, Bt, be):
    nspec = pl.BlockSpec((BN, HIDDEN), lambda i: (i, 0))
    return pl.pallas_call(
        _tc_final_body,
        grid=(GRID_N,),
        in_specs=[
            pl.BlockSpec((BN, D_NODE), lambda i: (i, 0)),
            nspec, nspec, nspec, nspec,
            pl.BlockSpec((BN, 1), lambda i: (i, 0)),
            pl.BlockSpec((D_NODE, HIDDEN), _wmap),
            pl.BlockSpec((HIDDEN, HIDDEN), _wmap),
            pl.BlockSpec((1, HIDDEN), _wmap),
        ],
        out_specs=pl.BlockSpec((N_GRAPHS, HIDDEN), lambda i: (0, 0)),
        out_shape=jax.ShapeDtypeStruct((N_GRAPHS, HIDDEN), jnp.float32),
    )(x, pa0, pa1, pb0, pb1, bt, At, Bt, be)


def kernel(x, edge_index, edge_attr, batch, W_init, b_init, W1, b1, W2, b2,
           W3, b3, W_e2n, b_e2n):
    row = edge_index[0].astype(jnp.int32)
    col = edge_index[1].astype(jnp.int32)
    pad = ((0, IDXPAD - IDXROWS), (0, 0))
    row2d = jnp.pad(row.reshape(IDXROWS, K), pad)
    col2d = jnp.pad(col.reshape(IDXROWS, K), pad)

    Wxt = W_init[:, :D_NODE].T
    Wet = W_init[:, D_NODE:].T
    Wts = (W1.T, W2.T, W3.T)
    bis = (b1[None, :], b2[None, :], b3[None, :])
    zrows = jnp.zeros((ZR, HIDDEN), jnp.float32)

    ea = edge_attr.astype(jnp.bfloat16)
    xg_a = _sc_gather(x, row2d, SEG_A)
    xg_b = _sc_gather(x, row2d, SEG_B)
    h0_a, y_lo = _tc_init(xg_a, ea, Wxt, Wet, b_init[None, :], Wts[0],
                          GRID_A, 0)
    h0_b, y_hi = _tc_init(xg_b, ea, Wxt, Wet, b_init[None, :], Wts[0],
                          GRID_B, GRID_A)

    h3_lo = h3_hi = None
    for i in range(3):
        part_a = _sc_scatter(y_lo, col2d, zrows, SEG_A)
        part_b = _sc_scatter(y_hi, col2d, zrows, SEG_B)
        agg = _tc_add4(part_a[0], part_a[1], part_b[0], part_b[1])
        g_a = _sc_gather(agg, row2d, SEG_A)
        g_b = _sc_gather(agg, row2d, SEG_B)
        if i < 2:
            y_lo = _tc_conv(g_a, y_lo, h0_a, bis[i], Wts[i + 1], GRID_A, 0)
            y_hi = _tc_conv(g_b, y_hi, h0_b, bis[i], Wts[i + 1], GRID_B, GRID_A)
        else:
            h3_lo = _tc_convlast(g_a, y_lo, h0_a, bis[i], GRID_A, 0)
            h3_hi = _tc_convlast(g_b, y_hi, h0_b, bis[i], GRID_B, GRID_A)

    part_a = _sc_scatter(h3_lo, col2d, zrows, SEG_A)
    part_b = _sc_scatter(h3_hi, col2d, zrows, SEG_B)
    bt = batch.astype(jnp.int32).reshape(N_NODES, 1)
    emb = _tc_final(x, part_a[0], part_a[1], part_b[0], part_b[1], bt,
                    W_e2n[:, :D_NODE].T, W_e2n[:, D_NODE:].T, b_e2n[None, :])
    return emb
